# baseline (speedup 1.0000x reference)
import functools

import jax
import jax.numpy as jnp
from jax import lax
from jax.experimental import pallas as pl
from jax.experimental.pallas import tpu as pltpu
from jax.experimental.pallas import tpu_sc as plsc

N = 10000
E = 320000
D = 128

NC, NS, L = 2, 16, 16
NW = NC * NS
S = 5
ES = E // S
ROWS_PER_W = (2 * ES) // NW
CHUNK = 80
N_CHUNKS = ROWS_PER_W // CHUNK


def _sc_gather_mul(srcs, dsts, x):
    mesh = plsc.VectorSubcoreMesh(core_axis_name="c", subcore_axis_name="s")
    f32 = jnp.float32

    @functools.partial(
        pl.kernel,
        out_type=jax.ShapeDtypeStruct((2 * ES, D), f32),
        mesh=mesh,
        scratch_types=[
            pltpu.VMEM((ROWS_PER_W,), jnp.int32),
            pltpu.VMEM((ROWS_PER_W,), jnp.int32),
            [pltpu.VMEM((CHUNK, D), f32)] * 2,
            [pltpu.VMEM((CHUNK, D), f32)] * 2,
            [pltpu.VMEM((CHUNK, D), f32)] * 2,
            [pltpu.SemaphoreType.DMA] * 2,
            [pltpu.SemaphoreType.DMA] * 2,
            pltpu.SemaphoreType.DMA,
        ],
    )
    def k(srcs_hbm, dsts_hbm, x_hbm, h_hbm, idx_s, idx_d, a, b, o,
          sem_g, sem_w, sem_i):
        wid = lax.axis_index("s") * NC + lax.axis_index("c")
        w_base = wid * ROWS_PER_W

        ci = pltpu.async_copy(srcs_hbm.at[pl.ds(w_base, ROWS_PER_W)], idx_s,
                              sem_i)
        cd = pltpu.async_copy(dsts_hbm.at[pl.ds(w_base, ROWS_PER_W)], idx_d,
                              sem_i)
        ci.wait()
        cd.wait()

        def fire(kk, p):
            pltpu.async_copy(x_hbm.at[idx_s.at[pl.ds(kk * CHUNK, CHUNK)]],
                             a[p], sem_g[p])
            pltpu.async_copy(x_hbm.at[idx_d.at[pl.ds(kk * CHUNK, CHUNK)]],
                             b[p], sem_g[p])

        fire(0, 0)

        @pl.loop(0, N_CHUNKS, step=2)
        def chunk_pair(k0):
            for p in range(2):
                kk = k0 + p

                @pl.when(kk + 1 < N_CHUNKS)
                def _():
                    fire(kk + 1, 1 - p)

                pltpu.make_async_copy(x_hbm.at[idx_s.at[pl.ds(0, CHUNK)]],
                                      a[p], sem_g[p]).wait()
                pltpu.make_async_copy(x_hbm.at[idx_d.at[pl.ds(0, CHUNK)]],
                                      b[p], sem_g[p]).wait()

                @pl.when(kk >= 2)
                def _():
                    pltpu.make_async_copy(
                        o[p], h_hbm.at[pl.ds(0, CHUNK)],
                        sem_w[p]).wait()

                @plsc.parallel_loop(0, CHUNK, 1, unroll=8)
                def row_body(r):
                    for j in range(D // L):
                        sl = pl.ds(j * L, L)
                        o[p][r, sl] = a[p][r, sl] * b[p][r, sl]

                pltpu.async_copy(
                    o[p], h_hbm.at[pl.ds(w_base + kk * CHUNK, CHUNK)],
                    sem_w[p])

        for p in range(2):
            pltpu.make_async_copy(o[p], h_hbm.at[pl.ds(0, CHUNK)],
                                  sem_w[p]).wait()

    return k(srcs, dsts, x)


B_TC = 12800
NBLK = ES // B_TC


def _tc_body(hp_ref, hn_ref, w1_ref, b1_ref, w2_ref, b2_ref, oT_ref):
    cat = jnp.concatenate(
        [jnp.maximum(hp_ref[...], 0.0), jnp.maximum(hn_ref[...], 0.0)], axis=1)
    z = jnp.maximum(
        jnp.dot(cat.astype(jnp.bfloat16), w1_ref[...],
                preferred_element_type=jnp.float32)
        + b1_ref[...], 0.0)
    o = jax.nn.sigmoid(
        jnp.dot(z, w2_ref[...], preferred_element_type=jnp.float32)
        + b2_ref[...])
    oT_ref[...] = o.T


def _tc_mlp(h2, Wb1, bb1, Wb2, bb2):
    return pl.pallas_call(
        _tc_body,
        grid=(NBLK,),
        in_specs=[
            pl.BlockSpec((B_TC, D), lambda i: (i, 0)),
            pl.BlockSpec((B_TC, D), lambda i: (i + NBLK, 0)),
            pl.BlockSpec((2 * D, 3 * D), lambda i: (0, 0)),
            pl.BlockSpec((1, 3 * D), lambda i: (0, 0)),
            pl.BlockSpec((3 * D, 8), lambda i: (0, 0)),
            pl.BlockSpec((1, 8), lambda i: (0, 0)),
        ],
        out_specs=pl.BlockSpec((8, B_TC), lambda i: (0, i)),
        out_shape=jax.ShapeDtypeStruct((8, ES), jnp.float32),
    )(h2, h2, Wb1, bb1, Wb2, bb2)


@jax.jit
def kernel(x, edge_index, edge_index_neg, W1, b1, W2, b2, We1, be1, We2, be2):
    f32 = jnp.float32
    Wb1 = jnp.zeros((2 * D, 3 * D), f32)
    Wb1 = Wb1.at[:D, :D].set(W1).at[:D, D:2 * D].set(We1).at[D:, 2 * D:].set(We1)
    Wb1 = Wb1.astype(jnp.bfloat16)
    bb1 = jnp.concatenate([b1, be1, be1]).reshape(1, 3 * D)
    Wb2 = jnp.zeros((3 * D, 8), f32)
    Wb2 = Wb2.at[:D, :4].set(W2).at[D:2 * D, 4:5].set(We2).at[2 * D:, 5:6].set(We2)
    bb2 = jnp.concatenate([b2, be2, be2, jnp.zeros((2,), f32)]).reshape(1, 8)

    oTs = []
    for i in range(S):
        sl = slice(i * ES, (i + 1) * ES)
        srcs = jnp.concatenate([edge_index[0, sl], edge_index_neg[0, sl]])
        dsts = jnp.concatenate([edge_index[1, sl], edge_index_neg[1, sl]])
        h2 = _sc_gather_mul(srcs, dsts, x)
        oTs.append(_tc_mlp(h2, Wb1, bb1, Wb2, bb2))
    oT = jnp.concatenate(oTs, axis=1)
    return oT[:4].T, oT[4], oT[5]


# speedup vs baseline: 1.1464x; 1.1464x over previous
import functools

import jax
import jax.numpy as jnp
from jax import lax
from jax.experimental import pallas as pl
from jax.experimental.pallas import tpu as pltpu
from jax.experimental.pallas import tpu_sc as plsc

N = 10000
E = 320000
D = 128

NC, NS, L = 2, 16, 16
NW = NC * NS
S = 5
ES = E // S
ROWS_PER_W = (2 * ES) // NW
CHUNK = 80
N_CHUNKS = ROWS_PER_W // CHUNK


def _sc_gather_mul(srcs, dsts, x):
    mesh = plsc.VectorSubcoreMesh(core_axis_name="c", subcore_axis_name="s")
    f32 = jnp.float32

    @functools.partial(
        pl.kernel,
        out_type=jax.ShapeDtypeStruct((ES, D), jnp.int32),
        mesh=mesh,
        scratch_types=[
            pltpu.VMEM((ROWS_PER_W,), jnp.int32),
            pltpu.VMEM((ROWS_PER_W,), jnp.int32),
            [pltpu.VMEM((CHUNK, D), f32)] * 2,
            [pltpu.VMEM((CHUNK, D), f32)] * 2,
            [pltpu.VMEM((CHUNK // 2, D), jnp.int32)] * 2,
            [pltpu.SemaphoreType.DMA] * 2,
            [pltpu.SemaphoreType.DMA] * 2,
            pltpu.SemaphoreType.DMA,
        ],
        compiler_params=pltpu.CompilerParams(needs_layout_passes=False),
    )
    def k(srcs_hbm, dsts_hbm, x_hbm, h_hbm, idx_s, idx_d, a, b, o,
          sem_g, sem_w, sem_i):
        wid = lax.axis_index("s") * NC + lax.axis_index("c")
        w_base = wid * ROWS_PER_W

        ci = pltpu.async_copy(srcs_hbm.at[pl.ds(w_base, ROWS_PER_W)], idx_s,
                              sem_i)
        cd = pltpu.async_copy(dsts_hbm.at[pl.ds(w_base, ROWS_PER_W)], idx_d,
                              sem_i)
        ci.wait()
        cd.wait()

        def fire(kk, p):
            pltpu.async_copy(x_hbm.at[idx_s.at[pl.ds(kk * CHUNK, CHUNK)]],
                             a[p], sem_g[p])
            pltpu.async_copy(x_hbm.at[idx_d.at[pl.ds(kk * CHUNK, CHUNK)]],
                             b[p], sem_g[p])

        fire(0, 0)

        @pl.loop(0, N_CHUNKS, step=2)
        def chunk_pair(k0):
            for p in range(2):
                kk = k0 + p

                @pl.when(kk + 1 < N_CHUNKS)
                def _():
                    fire(kk + 1, 1 - p)

                pltpu.make_async_copy(x_hbm.at[idx_s.at[pl.ds(0, CHUNK)]],
                                      a[p], sem_g[p]).wait()
                pltpu.make_async_copy(x_hbm.at[idx_d.at[pl.ds(0, CHUNK)]],
                                      b[p], sem_g[p]).wait()

                @pl.when(kk >= 2)
                def _():
                    pltpu.make_async_copy(
                        o[p], h_hbm.at[pl.ds(0, CHUNK // 2)],
                        sem_w[p]).wait()

                @plsc.parallel_loop(0, CHUNK // 2, 1, unroll=4)
                def pair_body(t):
                    for j in range(D // L):
                        sl = pl.ds(j * L, L)
                        p0 = a[p][2 * t, sl] * b[p][2 * t, sl]
                        p1 = a[p][2 * t + 1, sl] * b[p][2 * t + 1, sl]
                        o[p][t, sl] = plsc.bitcast(
                            plsc.pack(p0, p1,
                                      format=plsc.PackFormat.INTERLEAVED),
                            jnp.int32)

                pltpu.async_copy(
                    o[p],
                    h_hbm.at[pl.ds(wid * (ROWS_PER_W // 2) + kk * (CHUNK // 2), CHUNK // 2)],
                    sem_w[p])

        for p in range(2):
            pltpu.make_async_copy(o[p], h_hbm.at[pl.ds(0, CHUNK // 2)],
                                  sem_w[p]).wait()

    return k(srcs, dsts, x)


B_TC = 12800
NBLK = ES // B_TC


def _tc_body(hp_ref, hn_ref, w1_ref, b1_ref, w2_ref, b2_ref, oT_ref):
    zero = jnp.bfloat16(0.0)
    hp = pltpu.bitcast(hp_ref[...], jnp.bfloat16)
    hn = pltpu.bitcast(hn_ref[...], jnp.bfloat16)
    cat = jnp.concatenate(
        [jnp.maximum(hp, zero), jnp.maximum(hn, zero)], axis=1)
    z = jnp.maximum(
        jnp.dot(cat, w1_ref[...], preferred_element_type=jnp.float32)
        + b1_ref[...], 0.0)
    o = jax.nn.sigmoid(
        jnp.dot(z, w2_ref[...], preferred_element_type=jnp.float32)
        + b2_ref[...])
    oT_ref[...] = o.T


def _tc_mlp(h2, Wb1, bb1, Wb2, bb2):
    return pl.pallas_call(
        _tc_body,
        grid=(NBLK,),
        in_specs=[
            pl.BlockSpec((B_TC // 2, D), lambda i: (i, 0)),
            pl.BlockSpec((B_TC // 2, D), lambda i: (i + NBLK, 0)),
            pl.BlockSpec((2 * D, 3 * D), lambda i: (0, 0)),
            pl.BlockSpec((1, 3 * D), lambda i: (0, 0)),
            pl.BlockSpec((3 * D, 8), lambda i: (0, 0)),
            pl.BlockSpec((1, 8), lambda i: (0, 0)),
        ],
        out_specs=pl.BlockSpec((8, B_TC), lambda i: (0, i)),
        out_shape=jax.ShapeDtypeStruct((8, ES), jnp.float32),
    )(h2, h2, Wb1, bb1, Wb2, bb2)


@jax.jit
def kernel(x, edge_index, edge_index_neg, W1, b1, W2, b2, We1, be1, We2, be2):
    f32 = jnp.float32
    Wb1 = jnp.zeros((2 * D, 3 * D), f32)
    Wb1 = Wb1.at[:D, :D].set(W1).at[:D, D:2 * D].set(We1).at[D:, 2 * D:].set(We1)
    Wb1 = Wb1.astype(jnp.bfloat16)
    bb1 = jnp.concatenate([b1, be1, be1]).reshape(1, 3 * D)
    Wb2 = jnp.zeros((3 * D, 8), f32)
    Wb2 = Wb2.at[:D, :4].set(W2).at[D:2 * D, 4:5].set(We2).at[2 * D:, 5:6].set(We2)
    bb2 = jnp.concatenate([b2, be2, be2, jnp.zeros((2,), f32)]).reshape(1, 8)

    oTs = []
    for i in range(S):
        sl = slice(i * ES, (i + 1) * ES)
        srcs = jnp.concatenate([edge_index[0, sl], edge_index_neg[0, sl]])
        dsts = jnp.concatenate([edge_index[1, sl], edge_index_neg[1, sl]])
        h2 = _sc_gather_mul(srcs, dsts, x)
        oTs.append(_tc_mlp(h2, Wb1, bb1, Wb2, bb2))
    oT = jnp.concatenate(oTs, axis=1)
    return oT[:4].T, oT[4], oT[5]


# speedup vs baseline: 1.1812x; 1.0303x over previous
import functools

import jax
import jax.numpy as jnp
from jax import lax
from jax.experimental import pallas as pl
from jax.experimental.pallas import tpu as pltpu
from jax.experimental.pallas import tpu_sc as plsc

N = 10000
E = 320000
D = 128

NC, NS, L = 2, 16, 16
NW = NC * NS
ES_PARTS = (76800, 76800, 76800, 76800, 12800)
CHUNK = 80


def _sc_gather_mul(srcs, dsts, x, es):
    mesh = plsc.VectorSubcoreMesh(core_axis_name="c", subcore_axis_name="s")
    f32 = jnp.float32
    ROWS_PER_W = (2 * es) // NW
    N_CHUNKS = ROWS_PER_W // CHUNK

    @functools.partial(
        pl.kernel,
        out_type=jax.ShapeDtypeStruct((es, D), jnp.int32),
        mesh=mesh,
        scratch_types=[
            pltpu.VMEM((ROWS_PER_W,), jnp.int32),
            pltpu.VMEM((ROWS_PER_W,), jnp.int32),
            [pltpu.VMEM((CHUNK, D), f32)] * 2,
            [pltpu.VMEM((CHUNK, D), f32)] * 2,
            [pltpu.VMEM((CHUNK // 2, D), jnp.int32)] * 2,
            [pltpu.SemaphoreType.DMA] * 2,
            [pltpu.SemaphoreType.DMA] * 2,
            pltpu.SemaphoreType.DMA,
        ],
        compiler_params=pltpu.CompilerParams(needs_layout_passes=False),
    )
    def k(srcs_hbm, dsts_hbm, x_hbm, h_hbm, idx_s, idx_d, a, b, o,
          sem_g, sem_w, sem_i):
        wid = lax.axis_index("s") * NC + lax.axis_index("c")
        w_base = wid * ROWS_PER_W

        ci = pltpu.async_copy(srcs_hbm.at[pl.ds(w_base, ROWS_PER_W)], idx_s,
                              sem_i)
        cd = pltpu.async_copy(dsts_hbm.at[pl.ds(w_base, ROWS_PER_W)], idx_d,
                              sem_i)
        ci.wait()
        cd.wait()

        def fire(kk, p):
            pltpu.async_copy(x_hbm.at[idx_s.at[pl.ds(kk * CHUNK, CHUNK)]],
                             a[p], sem_g[p])
            pltpu.async_copy(x_hbm.at[idx_d.at[pl.ds(kk * CHUNK, CHUNK)]],
                             b[p], sem_g[p])

        fire(0, 0)

        @pl.loop(0, N_CHUNKS, step=2)
        def chunk_pair(k0):
            for p in range(2):
                kk = k0 + p

                @pl.when(kk + 1 < N_CHUNKS)
                def _():
                    fire(kk + 1, 1 - p)

                pltpu.make_async_copy(x_hbm.at[idx_s.at[pl.ds(0, CHUNK)]],
                                      a[p], sem_g[p]).wait()
                pltpu.make_async_copy(x_hbm.at[idx_d.at[pl.ds(0, CHUNK)]],
                                      b[p], sem_g[p]).wait()

                @pl.when(kk >= 2)
                def _():
                    pltpu.make_async_copy(
                        o[p], h_hbm.at[pl.ds(0, CHUNK // 2)],
                        sem_w[p]).wait()

                @plsc.parallel_loop(0, CHUNK // 2, 1, unroll=4)
                def pair_body(t):
                    for j in range(D // L):
                        sl = pl.ds(j * L, L)
                        p0 = a[p][2 * t, sl] * b[p][2 * t, sl]
                        p1 = a[p][2 * t + 1, sl] * b[p][2 * t + 1, sl]
                        o[p][t, sl] = plsc.bitcast(
                            plsc.pack(p0, p1,
                                      format=plsc.PackFormat.INTERLEAVED),
                            jnp.int32)

                pltpu.async_copy(
                    o[p],
                    h_hbm.at[pl.ds(wid * (ROWS_PER_W // 2) + kk * (CHUNK // 2), CHUNK // 2)],
                    sem_w[p])

        for p in range(2):
            pltpu.make_async_copy(o[p], h_hbm.at[pl.ds(0, CHUNK // 2)],
                                  sem_w[p]).wait()

    return k(srcs, dsts, x)


B_TC = 12800


def _tc_body(hp_ref, hn_ref, w1_ref, b1_ref, w2_ref, b2_ref, oT_ref):
    zero = jnp.bfloat16(0.0)
    hp = pltpu.bitcast(hp_ref[...], jnp.bfloat16)
    hn = pltpu.bitcast(hn_ref[...], jnp.bfloat16)
    cat = jnp.concatenate(
        [jnp.maximum(hp, zero), jnp.maximum(hn, zero)], axis=1)
    z = jnp.maximum(
        jnp.dot(cat, w1_ref[...], preferred_element_type=jnp.float32)
        + b1_ref[...], 0.0)
    o = jax.nn.sigmoid(
        jnp.dot(z, w2_ref[...], preferred_element_type=jnp.float32)
        + b2_ref[...])
    oT_ref[...] = o.T


def _tc_mlp(h2, Wb1, bb1, Wb2, bb2, es):
    nblk = es // B_TC
    return pl.pallas_call(
        _tc_body,
        grid=(nblk,),
        in_specs=[
            pl.BlockSpec((B_TC // 2, D), lambda i: (i, 0)),
            pl.BlockSpec((B_TC // 2, D), lambda i, n=nblk: (i + n, 0)),
            pl.BlockSpec((2 * D, 3 * D), lambda i: (0, 0)),
            pl.BlockSpec((1, 3 * D), lambda i: (0, 0)),
            pl.BlockSpec((3 * D, 8), lambda i: (0, 0)),
            pl.BlockSpec((1, 8), lambda i: (0, 0)),
        ],
        out_specs=pl.BlockSpec((8, B_TC), lambda i: (0, i)),
        out_shape=jax.ShapeDtypeStruct((8, es), jnp.float32),
    )(h2, h2, Wb1, bb1, Wb2, bb2)


@jax.jit
def kernel(x, edge_index, edge_index_neg, W1, b1, W2, b2, We1, be1, We2, be2):
    f32 = jnp.float32
    Wb1 = jnp.zeros((2 * D, 3 * D), f32)
    Wb1 = Wb1.at[:D, :D].set(W1).at[:D, D:2 * D].set(We1).at[D:, 2 * D:].set(We1)
    Wb1 = Wb1.astype(jnp.bfloat16)
    bb1 = jnp.concatenate([b1, be1, be1]).reshape(1, 3 * D)
    Wb2 = jnp.zeros((3 * D, 8), f32)
    Wb2 = Wb2.at[:D, :4].set(W2).at[D:2 * D, 4:5].set(We2).at[2 * D:, 5:6].set(We2)
    bb2 = jnp.concatenate([b2, be2, be2, jnp.zeros((2,), f32)]).reshape(1, 8)

    oTs = []
    e0 = 0
    for es in ES_PARTS:
        sl = slice(e0, e0 + es)
        e0 += es
        srcs = jnp.concatenate([edge_index[0, sl], edge_index_neg[0, sl]])
        dsts = jnp.concatenate([edge_index[1, sl], edge_index_neg[1, sl]])
        h2 = _sc_gather_mul(srcs, dsts, x, es)
        oTs.append(_tc_mlp(h2, Wb1, bb1, Wb2, bb2, es))
    oT = jnp.concatenate(oTs, axis=1)
    return oT[:4].T, oT[4], oT[5]
